# baseline (speedup 1.0000x reference)
import functools

import jax
import jax.numpy as jnp
from jax import lax
from jax.experimental import pallas as pl
from jax.experimental.pallas import tpu as pltpu
from jax.experimental.pallas import tpu_sc as plsc

N = 100000
NP = 102400
E = 1600000
ROWS = 12800
ROWSP = ROWS + 16
EP = ROWSP * 128
C = NP // 2
DUMMY = 400
NS = 16
FW = 16

f32 = jnp.float32
i32 = jnp.int32

_mesh = plsc.VectorSubcoreMesh(core_axis_name="c", subcore_axis_name="s")
_sc_params = pltpu.CompilerParams(use_tc_tiling_on_sc=False)


def _deg_body(dst_hbm, w_hbm, deg_out, dl_out, we_out,
              dst_v, w_v, dl0_v, dl1_v, w0_v, w1_v, zb, acc_sh):
    c = lax.axis_index("c")
    s = lax.axis_index("s")
    widx = c * NS + s
    iota = lax.iota(i32, 16)

    def zfill(i, _):
        zb[pl.ds(i * 16, 16)] = jnp.zeros((16,), f32)
        return 0

    lax.fori_loop(0, 6400 // 16, zfill, 0)
    pltpu.sync_copy(zb, acc_sh.at[pl.ds(s * 6400, 6400)])
    plsc.subcore_barrier()

    base = widx * 400

    def stage(it, _):
        pltpu.sync_copy(dst_hbm.at[pl.ds(base + it * 16, 16), :], dst_v)
        pltpu.sync_copy(w_hbm.at[pl.ds(base + it * 16, 16), :], w_v)
        for j in range(16):
            pltpu.sync_copy(w_v.at[j], acc_sh.at[dst_v.at[j]], add=True)

        def prep(j, _):
            for k in range(8):
                sl16 = pl.ds(k * 16, 16)
                d = dst_v[j, sl16]
                wv = w_v[j, sl16]
                dummy = C + k * 16 + iota
                m0 = d < C
                dl0_v[j, sl16] = jnp.where(m0, d, dummy)
                w0_v[j, sl16] = jnp.where(m0, wv, 0.0)
                d1 = d - C
                m1 = d1 >= 0
                dl1_v[j, sl16] = jnp.where(m1, d1, dummy)
                w1_v[j, sl16] = jnp.where(m1, wv, 0.0)
            return 0

        lax.fori_loop(0, 16, prep, 0)
        rr = pl.ds(base + it * 16, 16)
        pltpu.sync_copy(dl0_v, dl_out.at[0, rr, :])
        pltpu.sync_copy(dl1_v, dl_out.at[1, rr, :])
        pltpu.sync_copy(w0_v, we_out.at[0, rr, :])
        pltpu.sync_copy(w1_v, we_out.at[1, rr, :])
        return 0

    lax.fori_loop(0, 25, stage, 0)
    plsc.subcore_barrier()
    pltpu.sync_copy(acc_sh.at[pl.ds(s * 6400, 6400)],
                    deg_out.at[c, pl.ds(s * 6400, 6400)])


_k_deg = functools.partial(
    pl.kernel,
    out_type=(
        jax.ShapeDtypeStruct((2, NP), f32),
        jax.ShapeDtypeStruct((2, ROWSP, 128), i32),
        jax.ShapeDtypeStruct((2, ROWSP, 128), f32),
    ),
    mesh=_mesh,
    compiler_params=_sc_params,
    scratch_types=[
        pltpu.VMEM((16, 128), i32),
        pltpu.VMEM((16, 128), f32),
        pltpu.VMEM((16, 128), i32),
        pltpu.VMEM((16, 128), i32),
        pltpu.VMEM((16, 128), f32),
        pltpu.VMEM((16, 128), f32),
        pltpu.VMEM((6400,), f32),
        pltpu.VMEM_SHARED((NP,), f32),
    ],
)(_deg_body)


def _dis_body(deg_hbm, dis_out, b0, b1):
    c = lax.axis_index("c")
    s = lax.axis_index("s")
    widx = c * NS + s
    sl = widx * 3200
    pltpu.sync_copy(deg_hbm.at[0, pl.ds(sl, 3200)], b0)
    pltpu.sync_copy(deg_hbm.at[1, pl.ds(sl, 3200)], b1)

    def body(i, _):
        ds16 = pl.ds(i * 16, 16)
        x = b0[ds16] + b1[ds16] + 1.0
        bits = lax.bitcast_convert_type(x, i32)
        y = lax.bitcast_convert_type(
            jnp.int32(0x5F3759DF) - lax.shift_right_logical(bits, 1), f32)
        for _ in range(4):
            y = y * (1.5 - 0.5 * x * y * y)
        b0[ds16] = y
        return 0

    lax.fori_loop(0, 200, body, 0)
    pltpu.sync_copy(b0, dis_out.at[pl.ds(sl, 3200)])


_k_dis = functools.partial(
    pl.kernel,
    out_type=jax.ShapeDtypeStruct((NP,), f32),
    mesh=_mesh,
    compiler_params=_sc_params,
    scratch_types=[
        pltpu.VMEM((3200,), f32),
        pltpu.VMEM((3200,), f32),
    ],
)(_dis_body)


def _layer_body(src_hbm, dl_hbm, we_hbm, g_hbm, dis_hbm, b_hbm, out_hbm,
                src_v, dst_v, w_v, rows_v, bias_v, acc_e, g_e, dis_e,
                acc_sh, sem_g, sem_st, sem_s):
    SR = 8
    c = lax.axis_index("c")
    s = lax.axis_index("s")
    nbase = c * C

    def zf(i, _):
        acc_e[i, :] = jnp.zeros((16,), f32)
        return 0

    lax.fori_loop(0, 400, zf, 0)
    for k in range(8):
        pltpu.sync_copy(acc_e, acc_sh.at[pl.ds(s * 3200 + k * 400, 400), :])

    @pl.when(s == 0)
    def _():
        pltpu.sync_copy(acc_e, acc_sh.at[pl.ds(C, DUMMY), :])

    pltpu.sync_copy(b_hbm, bias_v)
    plsc.subcore_barrier()

    r_base = s * 800

    def _stage(n, pp, copy):
        rr = r_base + n * SR
        copy(src_hbm.at[pl.ds(rr, SR), :], src_v.at[pp], sem_st)
        copy(dl_hbm.at[c, pl.ds(rr, SR), :], dst_v.at[pp], sem_st)
        copy(we_hbm.at[c, pl.ds(rr, SR), :], w_v.at[pp], sem_st)

    def _fire_gathers(pp):
        for j in range(SR):
            pltpu.async_copy(g_hbm.at[src_v.at[pp, j]], rows_v.at[pp, j],
                             sem_g.at[pp, j])

    _stage(0, 0, lambda a, b, sem: pltpu.sync_copy(a, b))
    _fire_gathers(0)

    def outer(it, _):
        for p in (0, 1):
            n = 2 * it + p
            q = 1 - p
            @pl.when(n > 0)
            def _():
                for j in range(SR):
                    pltpu.make_async_copy(
                        rows_v.at[q, j], acc_sh.at[pl.ds(0, 128), :],
                        sem_s).wait()

            _stage(n + 1, q, lambda a, b, sem: pltpu.async_copy(a, b, sem))
            for j in range(SR):
                pltpu.make_async_copy(
                    g_hbm.at[src_v.at[p, j]], rows_v.at[p, j],
                    sem_g.at[p, j]).wait()

                def scale(k, _):
                    wg = w_v[p, j, pl.ds(k * 16, 16)]
                    for l in range(16):
                        e = k * 16 + l
                        rows_v[p, j, e, :] = rows_v[p, j, e, :] * wg[l]
                    return 0

                lax.fori_loop(0, 8, scale, 0, unroll=4)
                pltpu.async_copy(rows_v.at[p, j], acc_sh.at[dst_v.at[p, j]],
                                 sem_s, add=True)
            _stage(n + 1, q,
                   lambda a, b, sem: pltpu.make_async_copy(a, b, sem).wait())
            _fire_gathers(q)
        return 0

    lax.fori_loop(0, 800 // SR // 2, outer, 0)
    for j in range(SR):
        pltpu.make_async_copy(rows_v.at[1, j], acc_sh.at[pl.ds(0, 128), :],
                              sem_s).wait()
    for j in range(SR):
        pltpu.make_async_copy(g_hbm.at[src_v.at[0, j]], rows_v.at[0, j],
                              sem_g.at[0, j]).wait()
    plsc.subcore_barrier()

    def ep(k2, _):
        nb = s * 3200 + k2 * 400
        pltpu.sync_copy(acc_sh.at[pl.ds(nb, 400), :], acc_e)
        pltpu.sync_copy(g_hbm.at[pl.ds(nbase + nb, 400), :], g_e)
        pltpu.sync_copy(dis_hbm.at[pl.ds(nbase + nb, 400)], dis_e)

        def nod(k, _):
            dg = dis_e[pl.ds(k * 16, 16)]
            for l in range(16):
                i = k * 16 + l
                v = (acc_e[i, :] + g_e[i, :]) * dg[l] + bias_v[:]
                acc_e[i, :] = jnp.maximum(v, 0.0)
            return 0

        lax.fori_loop(0, 25, nod, 0)
        pltpu.sync_copy(acc_e, out_hbm.at[pl.ds(nbase + nb, 400), :])
        return 0

    lax.fori_loop(0, 8, ep, 0)


_k_layer = functools.partial(
    pl.kernel,
    out_type=jax.ShapeDtypeStruct((NP, FW), f32),
    mesh=_mesh,
    compiler_params=_sc_params,
    scratch_types=[
        pltpu.VMEM((2, 8, 128), i32),
        pltpu.VMEM((2, 8, 128), i32),
        pltpu.VMEM((2, 8, 128), f32),
        pltpu.VMEM((2, 8, 128, FW), f32),
        pltpu.VMEM((FW,), f32),
        pltpu.VMEM((400, FW), f32),
        pltpu.VMEM((400, FW), f32),
        pltpu.VMEM((400,), f32),
        pltpu.VMEM_SHARED((C + DUMMY, FW), f32),
        pltpu.SemaphoreType.DMA((2, 8)),
        pltpu.SemaphoreType.DMA,
        pltpu.SemaphoreType.DMA,
    ],
)(_layer_body)


_BLK = 6400


def _tc_g(a_parts, W, dis_col):
    n_in = len(a_parts)
    widths = [int(a.shape[1]) for a in a_parts]
    fout = int(W.shape[1])
    n_out = fout // 16

    def body(*refs):
        a_refs = refs[:n_in]
        w_ref = refs[n_in]
        d_ref = refs[n_in + 1]
        o_refs = refs[n_in + 2:]
        acc = None
        off = 0
        for ar, wd in zip(a_refs, widths):
            t = ar[...] @ w_ref[off:off + wd, :]
            acc = t if acc is None else acc + t
            off += wd
        g = acc * d_ref[...]
        for k, o in enumerate(o_refs):
            o[...] = g[:, k * 16:(k + 1) * 16]

    return pl.pallas_call(
        body,
        grid=(NP // _BLK,),
        in_specs=[pl.BlockSpec((_BLK, wd), lambda i: (i, 0)) for wd in widths]
        + [
            pl.BlockSpec((sum(widths), fout), lambda i: (0, 0)),
            pl.BlockSpec((_BLK, 1), lambda i: (i, 0)),
        ],
        out_specs=[pl.BlockSpec((_BLK, 16), lambda i: (i, 0))
                   for _ in range(n_out)],
        out_shape=[jax.ShapeDtypeStruct((NP, 16), f32) for _ in range(n_out)],
    )(*a_parts, W, dis_col)


def _tc_fin(h_parts, Wl, bl):
    n_in = len(h_parts)

    def body(*refs):
        a_refs = refs[:n_in]
        w_ref = refs[n_in]
        b_ref = refs[n_in + 1]
        o_ref = refs[n_in + 2]
        acc = None
        for k, ar in enumerate(a_refs):
            t = ar[...] @ w_ref[k * 16:(k + 1) * 16, :]
            acc = t if acc is None else acc + t
        o_ref[...] = acc + b_ref[0, 0]

    return pl.pallas_call(
        body,
        grid=(NP // _BLK,),
        in_specs=[pl.BlockSpec((_BLK, 16), lambda i: (i, 0))
                  for _ in range(n_in)]
        + [
            pl.BlockSpec((16 * n_in, 1), lambda i: (0, 0)),
            pl.BlockSpec((1, 1), lambda i: (0, 0)),
        ],
        out_specs=pl.BlockSpec((_BLK, 1), lambda i: (i, 0)),
        out_shape=jax.ShapeDtypeStruct((NP, 1), f32),
    )(*h_parts, Wl, bl.reshape(1, 1))


def kernel(x, edge_index, edge_weight, W1, b1, W2, b2, W3, b3, Wl, bl):
    src = edge_index[0].astype(i32)
    dst = edge_index[1].astype(i32)
    srcp = jnp.pad(src, (0, EP - E)).reshape(ROWSP, 128)
    dstp = jnp.pad(dst, (0, EP - E)).reshape(ROWSP, 128)
    wp = jnp.pad(edge_weight, (0, EP - E)).reshape(ROWSP, 128)
    xp = jnp.pad(x, ((0, NP - N), (0, 0)))

    deg2, dl2, we2 = _k_deg(dstp, wp)
    dis = _k_dis(deg2)
    dis_col = dis.reshape(NP, 1)

    def prop(g_parts, biases):
        return [
            _k_layer(srcp, dl2, we2, g, dis, b)
            for g, b in zip(g_parts, biases)
        ]

    g1 = _tc_g([xp], W1, dis_col)
    h1 = prop(g1, [b1])
    g2 = _tc_g(h1, W2, dis_col)
    h2 = prop(g2, [b2[:16], b2[16:]])
    g3 = _tc_g(h2, W3, dis_col)
    h3 = prop(g3, [b3[k * 16:(k + 1) * 16] for k in range(4)])
    out = _tc_fin(h3, Wl, bl)
    return out[:N]


# speedup vs baseline: 1.0309x; 1.0309x over previous
import functools

import jax
import jax.numpy as jnp
from jax import lax
from jax.experimental import pallas as pl
from jax.experimental.pallas import tpu as pltpu
from jax.experimental.pallas import tpu_sc as plsc

N = 100000
NP = 102400
E = 1600000
ROWS = 12800
ROWSP = ROWS + 16
EP = ROWSP * 128
C = NP // 2
DUMMY = 400
NS = 16
FW = 16

f32 = jnp.float32
i32 = jnp.int32

_mesh = plsc.VectorSubcoreMesh(core_axis_name="c", subcore_axis_name="s")
_sc_params = pltpu.CompilerParams(use_tc_tiling_on_sc=False)


def _deg_body(dst_hbm, w_hbm, deg_out, dst_v, w_v, zb, acc_sh):
    c = lax.axis_index("c")
    s = lax.axis_index("s")
    widx = c * NS + s

    def zfill(i, _):
        zb[pl.ds(i * 16, 16)] = jnp.zeros((16,), f32)
        return 0

    lax.fori_loop(0, 6400 // 16, zfill, 0)
    pltpu.sync_copy(zb, acc_sh.at[pl.ds(s * 6400, 6400)])
    plsc.subcore_barrier()

    base = widx * 400

    def stage(it, _):
        pltpu.sync_copy(dst_hbm.at[pl.ds(base + it * 16, 16), :], dst_v)
        pltpu.sync_copy(w_hbm.at[pl.ds(base + it * 16, 16), :], w_v)
        for j in range(16):
            pltpu.sync_copy(w_v.at[j], acc_sh.at[dst_v.at[j]], add=True)
        return 0

    lax.fori_loop(0, 25, stage, 0)
    plsc.subcore_barrier()
    pltpu.sync_copy(acc_sh.at[pl.ds(s * 6400, 6400)],
                    deg_out.at[c, pl.ds(s * 6400, 6400)])


_k_deg = functools.partial(
    pl.kernel,
    out_type=jax.ShapeDtypeStruct((2, NP), f32),
    mesh=_mesh,
    compiler_params=_sc_params,
    scratch_types=[
        pltpu.VMEM((16, 128), i32),
        pltpu.VMEM((16, 128), f32),
        pltpu.VMEM((6400,), f32),
        pltpu.VMEM_SHARED((NP,), f32),
    ],
)(_deg_body)


def _dis_body(deg_hbm, dis_out, b0, b1):
    c = lax.axis_index("c")
    s = lax.axis_index("s")
    widx = c * NS + s
    sl = widx * 3200
    pltpu.sync_copy(deg_hbm.at[0, pl.ds(sl, 3200)], b0)
    pltpu.sync_copy(deg_hbm.at[1, pl.ds(sl, 3200)], b1)

    def body(i, _):
        ds16 = pl.ds(i * 16, 16)
        x = b0[ds16] + b1[ds16] + 1.0
        bits = lax.bitcast_convert_type(x, i32)
        y = lax.bitcast_convert_type(
            jnp.int32(0x5F3759DF) - lax.shift_right_logical(bits, 1), f32)
        for _ in range(4):
            y = y * (1.5 - 0.5 * x * y * y)
        b0[ds16] = y
        return 0

    lax.fori_loop(0, 200, body, 0)
    pltpu.sync_copy(b0, dis_out.at[pl.ds(sl, 3200)])


_k_dis = functools.partial(
    pl.kernel,
    out_type=jax.ShapeDtypeStruct((NP,), f32),
    mesh=_mesh,
    compiler_params=_sc_params,
    scratch_types=[
        pltpu.VMEM((3200,), f32),
        pltpu.VMEM((3200,), f32),
    ],
)(_dis_body)


def _layer_body(src_hbm, dst_hbm, w_hbm, g_hbm, dis_hbm, b_hbm, out_hbm,
                src_v, dst_v, w_v, rows_v, bias_v, acc_e, g_e, dis_e,
                acc_sh, sem_g, sem_st, sem_s):
    SR = 8
    c = lax.axis_index("c")
    s = lax.axis_index("s")
    nbase = c * C

    def zf(i, _):
        acc_e[i, :] = jnp.zeros((16,), f32)
        return 0

    lax.fori_loop(0, 400, zf, 0)
    for k in range(8):
        pltpu.sync_copy(acc_e, acc_sh.at[pl.ds(s * 3200 + k * 400, 400), :])

    @pl.when(s == 0)
    def _():
        pltpu.sync_copy(acc_e, acc_sh.at[pl.ds(C, DUMMY), :])

    pltpu.sync_copy(b_hbm, bias_v)
    plsc.subcore_barrier()

    iota = lax.iota(i32, 16)
    r_base = s * 800

    def _stage(n, pp, copy):
        rr = r_base + n * SR
        for hbm, vm in ((src_hbm, src_v), (dst_hbm, dst_v), (w_hbm, w_v)):
            copy(hbm.at[pl.ds(rr, SR), :], vm.at[pp], sem_st)

    def _fire_gathers(pp):
        for j in range(SR):
            pltpu.async_copy(g_hbm.at[src_v.at[pp, j]], rows_v.at[pp, j],
                             sem_g.at[pp, j])

    rr0 = r_base
    for hbm, vm in ((src_hbm, src_v), (dst_hbm, dst_v), (w_hbm, w_v)):
        pltpu.sync_copy(hbm.at[pl.ds(rr0, SR), :], vm.at[0])
    _fire_gathers(0)

    def outer(it, _):
        for p in (0, 1):
            n = 2 * it + p
            q = 1 - p
            @pl.when(n > 0)
            def _():
                for j in range(SR):
                    pltpu.make_async_copy(
                        rows_v.at[q, j], acc_sh.at[pl.ds(0, 128), :],
                        sem_s).wait()

            _stage(n + 1, q, lambda a, b, sem: pltpu.async_copy(a, b, sem))
            def mk(jj, _):
                for k in range(8):
                    sl16 = pl.ds(k * 16, 16)
                    dl = dst_v[p, jj, sl16] - nbase
                    m = (dl >= 0) & (dl < C)
                    w_v[p, jj, sl16] = jnp.where(m, w_v[p, jj, sl16], 0.0)
                    dst_v[p, jj, sl16] = jnp.where(m, dl, C + k * 16 + iota)
                return 0

            lax.fori_loop(0, SR, mk, 0)
            for j in range(SR):
                pltpu.make_async_copy(
                    g_hbm.at[src_v.at[p, j]], rows_v.at[p, j],
                    sem_g.at[p, j]).wait()

                def scale(k, _):
                    wg = w_v[p, j, pl.ds(k * 16, 16)]
                    for l in range(16):
                        e = k * 16 + l
                        rows_v[p, j, e, :] = rows_v[p, j, e, :] * wg[l]
                    return 0

                lax.fori_loop(0, 8, scale, 0, unroll=4)
                pltpu.async_copy(rows_v.at[p, j], acc_sh.at[dst_v.at[p, j]],
                                 sem_s, add=True)
            _stage(n + 1, q,
                   lambda a, b, sem: pltpu.make_async_copy(a, b, sem).wait())
            _fire_gathers(q)
        return 0

    lax.fori_loop(0, 800 // SR // 2, outer, 0)
    for j in range(SR):
        pltpu.make_async_copy(rows_v.at[1, j], acc_sh.at[pl.ds(0, 128), :],
                              sem_s).wait()
    for j in range(SR):
        pltpu.make_async_copy(g_hbm.at[src_v.at[0, j]], rows_v.at[0, j],
                              sem_g.at[0, j]).wait()
    plsc.subcore_barrier()

    def ep(k2, _):
        nb = s * 3200 + k2 * 400
        pltpu.sync_copy(acc_sh.at[pl.ds(nb, 400), :], acc_e)
        pltpu.sync_copy(g_hbm.at[pl.ds(nbase + nb, 400), :], g_e)
        pltpu.sync_copy(dis_hbm.at[pl.ds(nbase + nb, 400)], dis_e)

        def nod(k, _):
            dg = dis_e[pl.ds(k * 16, 16)]
            for l in range(16):
                i = k * 16 + l
                v = (acc_e[i, :] + g_e[i, :]) * dg[l] + bias_v[:]
                acc_e[i, :] = jnp.maximum(v, 0.0)
            return 0

        lax.fori_loop(0, 25, nod, 0)
        pltpu.sync_copy(acc_e, out_hbm.at[pl.ds(nbase + nb, 400), :])
        return 0

    lax.fori_loop(0, 8, ep, 0)


_k_layer = functools.partial(
    pl.kernel,
    out_type=jax.ShapeDtypeStruct((NP, FW), f32),
    mesh=_mesh,
    compiler_params=_sc_params,
    scratch_types=[
        pltpu.VMEM((2, 8, 128), i32),
        pltpu.VMEM((2, 8, 128), i32),
        pltpu.VMEM((2, 8, 128), f32),
        pltpu.VMEM((2, 8, 128, FW), f32),
        pltpu.VMEM((FW,), f32),
        pltpu.VMEM((400, FW), f32),
        pltpu.VMEM((400, FW), f32),
        pltpu.VMEM((400,), f32),
        pltpu.VMEM_SHARED((C + DUMMY, FW), f32),
        pltpu.SemaphoreType.DMA((2, 8)),
        pltpu.SemaphoreType.DMA,
        pltpu.SemaphoreType.DMA,
    ],
)(_layer_body)


_BLK = 6400


def _tc_g(a_parts, W, dis_col):
    n_in = len(a_parts)
    widths = [int(a.shape[1]) for a in a_parts]
    fout = int(W.shape[1])
    n_out = fout // 16

    def body(*refs):
        a_refs = refs[:n_in]
        w_ref = refs[n_in]
        d_ref = refs[n_in + 1]
        o_refs = refs[n_in + 2:]
        acc = None
        off = 0
        for ar, wd in zip(a_refs, widths):
            t = ar[...] @ w_ref[off:off + wd, :]
            acc = t if acc is None else acc + t
            off += wd
        g = acc * d_ref[...]
        for k, o in enumerate(o_refs):
            o[...] = g[:, k * 16:(k + 1) * 16]

    return pl.pallas_call(
        body,
        grid=(NP // _BLK,),
        in_specs=[pl.BlockSpec((_BLK, wd), lambda i: (i, 0)) for wd in widths]
        + [
            pl.BlockSpec((sum(widths), fout), lambda i: (0, 0)),
            pl.BlockSpec((_BLK, 1), lambda i: (i, 0)),
        ],
        out_specs=[pl.BlockSpec((_BLK, 16), lambda i: (i, 0))
                   for _ in range(n_out)],
        out_shape=[jax.ShapeDtypeStruct((NP, 16), f32) for _ in range(n_out)],
    )(*a_parts, W, dis_col)


def _tc_fin(h_parts, Wl, bl):
    n_in = len(h_parts)

    def body(*refs):
        a_refs = refs[:n_in]
        w_ref = refs[n_in]
        b_ref = refs[n_in + 1]
        o_ref = refs[n_in + 2]
        acc = None
        for k, ar in enumerate(a_refs):
            t = ar[...] @ w_ref[k * 16:(k + 1) * 16, :]
            acc = t if acc is None else acc + t
        o_ref[...] = acc + b_ref[0, 0]

    return pl.pallas_call(
        body,
        grid=(NP // _BLK,),
        in_specs=[pl.BlockSpec((_BLK, 16), lambda i: (i, 0))
                  for _ in range(n_in)]
        + [
            pl.BlockSpec((16 * n_in, 1), lambda i: (0, 0)),
            pl.BlockSpec((1, 1), lambda i: (0, 0)),
        ],
        out_specs=pl.BlockSpec((_BLK, 1), lambda i: (i, 0)),
        out_shape=jax.ShapeDtypeStruct((NP, 1), f32),
    )(*h_parts, Wl, bl.reshape(1, 1))


def kernel(x, edge_index, edge_weight, W1, b1, W2, b2, W3, b3, Wl, bl):
    src = edge_index[0].astype(i32)
    dst = edge_index[1].astype(i32)
    srcp = jnp.pad(src, (0, EP - E)).reshape(ROWSP, 128)
    dstp = jnp.pad(dst, (0, EP - E)).reshape(ROWSP, 128)
    wp = jnp.pad(edge_weight, (0, EP - E)).reshape(ROWSP, 128)
    xp = jnp.pad(x, ((0, NP - N), (0, 0)))

    deg2 = _k_deg(dstp, wp)
    dis = _k_dis(deg2)
    dis_col = dis.reshape(NP, 1)

    def prop(g_parts, biases):
        return [
            _k_layer(srcp, dstp, wp, g, dis, b)
            for g, b in zip(g_parts, biases)
        ]

    g1 = _tc_g([xp], W1, dis_col)
    h1 = prop(g1, [b1])
    g2 = _tc_g(h1, W2, dis_col)
    h2 = prop(g2, [b2[:16], b2[16:]])
    g3 = _tc_g(h2, W3, dis_col)
    h3 = prop(g3, [b3[k * 16:(k + 1) * 16] for k in range(4)])
    out = _tc_fin(h3, Wl, bl)
    return out[:N]
